# baseline (speedup 1.0000x reference)
import jax
import jax.numpy as jnp
from jax import lax
from jax.experimental import pallas as pl
from jax.experimental.pallas import tpu as pltpu
from jax.experimental.pallas import tpu_sc as plsc

_NC = 2
_NS = 16
_NW = _NC * _NS
_L = 16


def _sc_mesh():
    return plsc.VectorSubcoreMesh(
        core_axis_name="c", subcore_axis_name="s",
        num_cores=_NC, num_subcores=_NS)


def _build_degrees(E, N):
    EP = E // _NW

    def body(src_hbm, dst_hbm, out_hbm, idx_a, idx_b, hist0, hist1,
             sema, semb):
        cid = lax.axis_index("c")
        sid = lax.axis_index("s")
        wid = sid * _NC + cid

        pltpu.async_copy(src_hbm.at[pl.ds(wid * EP, EP)], idx_a, sema)
        pltpu.async_copy(dst_hbm.at[pl.ds(wid * EP, EP)], idx_b, semb)

        def zero(i, _):
            hist0[pl.ds(i * _L, _L)] = jnp.zeros((_L,), jnp.float32)
            hist1[pl.ds(i * _L, _L)] = jnp.zeros((_L,), jnp.float32)
            return 0
        lax.fori_loop(0, N // _L, zero, 0)

        ones = jnp.ones((_L,), jnp.float32)
        pltpu.make_async_copy(src_hbm.at[pl.ds(wid * EP, EP)], idx_a,
                              sema).wait()

        def acc0(i, _):
            idx = idx_a[pl.ds(i * _L, _L)]
            plsc.addupdate_scatter(hist0, [idx], ones)
            return 0
        lax.fori_loop(0, EP // _L, acc0, 0)

        pltpu.make_async_copy(dst_hbm.at[pl.ds(wid * EP, EP)], idx_b,
                              semb).wait()

        def acc1(i, _):
            idx = idx_b[pl.ds(i * _L, _L)]
            plsc.addupdate_scatter(hist1, [idx], ones)
            return 0
        lax.fori_loop(0, EP // _L, acc1, 0)

        pltpu.sync_copy(hist0, out_hbm.at[0, wid])
        pltpu.sync_copy(hist1, out_hbm.at[1, wid])

    return pl.kernel(
        body,
        out_type=jax.ShapeDtypeStruct((2, _NW, N), jnp.float32),
        mesh=_sc_mesh(),
        compiler_params=pltpu.CompilerParams(needs_layout_passes=False, use_tc_tiling_on_sc=False),
        scratch_types=[
            pltpu.VMEM((EP,), jnp.int32),
            pltpu.VMEM((EP,), jnp.int32),
            pltpu.VMEM((N,), jnp.float32),
            pltpu.VMEM((N,), jnp.float32),
            pltpu.SemaphoreType.DMA,
            pltpu.SemaphoreType.DMA,
        ])


def _build_agg(N, D, E, K):
    EP = E // _NW
    NCH = EP // K
    RPT = N // _NS

    def body(h_hbm, src_hbm, dst_hbm, out_hbm, sidx, didx, gbuf,
             acc_sh, gsem, ssem):
        cid = lax.axis_index("c")
        sid = lax.axis_index("s")
        wid = sid * _NC + cid

        def z(i, _):
            for bk in range(2):
                for j in range(D // _L):
                    gbuf[bk, i, pl.ds(j * _L, _L)] = jnp.zeros(
                        (_L,), jnp.float32)
            return 0
        lax.fori_loop(0, K, z, 0)
        nfull = RPT // K
        for j in range(nfull):
            pltpu.sync_copy(gbuf.at[0], acc_sh.at[pl.ds(sid * RPT + j * K, K)])
        rem = RPT - nfull * K
        if rem:
            pltpu.sync_copy(gbuf.at[0, pl.ds(0, rem)],
                            acc_sh.at[pl.ds(sid * RPT + nfull * K, rem)])
        plsc.subcore_barrier()

        pltpu.sync_copy(src_hbm.at[wid], sidx)
        pltpu.sync_copy(dst_hbm.at[wid], didx)

        pltpu.async_copy(h_hbm.at[sidx.at[0]], gbuf.at[0], gsem.at[0])

        def step(c, _):
            p = lax.rem(c, 3)
            q = lax.rem(c + 1, 3)

            @pl.when(c >= 2)
            def _():
                pltpu.make_async_copy(
                    gbuf.at[q], acc_sh.at[didx.at[c - 2]], ssem.at[q]).wait()

            @pl.when(c + 1 < NCH)
            def _():
                pltpu.async_copy(h_hbm.at[sidx.at[c + 1]], gbuf.at[q],
                                 gsem.at[q])

            pltpu.make_async_copy(h_hbm.at[sidx.at[c]], gbuf.at[p],
                                  gsem.at[p]).wait()
            pltpu.async_copy(gbuf.at[p], acc_sh.at[didx.at[c]], ssem.at[p],
                             add=True)
            return 0
        lax.fori_loop(0, NCH, step, 0)
        for t in (NCH - 2, NCH - 1):
            pltpu.make_async_copy(gbuf.at[t % 3],
                                  acc_sh.at[didx.at[t]],
                                  ssem.at[t % 3]).wait()

        plsc.subcore_barrier()
        pltpu.sync_copy(acc_sh.at[pl.ds(sid * RPT, RPT)],
                        out_hbm.at[cid, pl.ds(sid * RPT, RPT)])

    return pl.kernel(
        body,
        out_type=jax.ShapeDtypeStruct((_NC, N, D), jnp.float32),
        mesh=_sc_mesh(),
        compiler_params=pltpu.CompilerParams(needs_layout_passes=False, use_tc_tiling_on_sc=False),
        scratch_types=[
            pltpu.VMEM((NCH, K), jnp.int32),
            pltpu.VMEM((NCH, K), jnp.int32),
            pltpu.VMEM((3, K, D), jnp.float32),
            pltpu.VMEM_SHARED((N, D), jnp.float32),
            pltpu.SemaphoreType.DMA((3,)),
            pltpu.SemaphoreType.DMA((3,)),
        ])


def _build_prep(N, D, BN):
    G = N // BN

    def body(degp_ref, x_ref, sel_ref, h_ref, nc_ref, nc_s):
        i = pl.program_id(0)

        @pl.when(i == 0)
        def _():
            dcols = lax.dot_general(
                degp_ref[...], sel_ref[...],
                (((0,), (0,)), ((), ())),
                preferred_element_type=jnp.float32)
            nc_s[...] = lax.rsqrt(jnp.where(dcols > 0, dcols, 1.0))

        nc_blk = nc_s[pl.ds(i * BN, BN), :]
        h_ref[...] = x_ref[...] * nc_blk[:, 0:1]
        nc_ref[...] = nc_blk

    return pl.pallas_call(
        body,
        grid=(G,),
        in_specs=[pl.BlockSpec((2 * _NW, N), lambda i: (0, 0)),
                  pl.BlockSpec((BN, D), lambda i: (i, 0)),
                  pl.BlockSpec((2 * _NW, 2), lambda i: (0, 0))],
        out_specs=[pl.BlockSpec((BN, D), lambda i: (i, 0)),
                   pl.BlockSpec((BN, 2), lambda i: (i, 0))],
        out_shape=[jax.ShapeDtypeStruct((N, D), jnp.float32),
                   jax.ShapeDtypeStruct((N, 2), jnp.float32)],
        scratch_shapes=[pltpu.VMEM((N, 2), jnp.float32)])


def _build_main(N, D, BN):
    G = N // BN

    inv_n = 1.0 / N

    def body(a0, a1, nc, x, w_ref, b_ref, wr_ref, br_ref, g_ref, bt_ref,
             out_ref, new_s, stats_s):
        ph = pl.program_id(0)
        j = pl.program_id(1)

        @pl.when(ph == 0)
        def _():
            agg = (a0[...] + a1[...]) * nc[:, 1:2]
            conv = jnp.maximum(
                jnp.dot(agg, w_ref[...], preferred_element_type=jnp.float32)
                + b_ref[...], 0.0)
            res = jnp.maximum(
                jnp.dot(x[...], wr_ref[...],
                        preferred_element_type=jnp.float32)
                + br_ref[...], 0.0)
            new = conv + res
            new_s[pl.ds(j * BN, BN), :] = new
            out_ref[...] = new
            blk = jnp.concatenate(
                [jnp.sum(new, axis=0, keepdims=True),
                 jnp.sum(new * new, axis=0, keepdims=True)], axis=0)

            @pl.when(j == 0)
            def _():
                stats_s[...] = blk

            @pl.when(j > 0)
            def _():
                stats_s[...] = stats_s[...] + blk

        @pl.when(ph == 1)
        def _():
            s = stats_s[...]
            mean = s[0:1, :] * inv_n
            var = s[1:2, :] * inv_n - mean * mean
            out_ref[...] = ((new_s[pl.ds(j * BN, BN), :] - mean)
                            * lax.rsqrt(var + 1e-5)
                            * g_ref[...] + bt_ref[...])

    return pl.pallas_call(
        body,
        grid=(2, G),
        in_specs=[pl.BlockSpec((BN, D), lambda p, i: (i, 0)),
                  pl.BlockSpec((BN, D), lambda p, i: (i, 0)),
                  pl.BlockSpec((BN, 2), lambda p, i: (i, 0)),
                  pl.BlockSpec((BN, D), lambda p, i: (i, 0)),
                  pl.BlockSpec((D, D), lambda p, i: (0, 0)),
                  pl.BlockSpec((1, D), lambda p, i: (0, 0)),
                  pl.BlockSpec((D, D), lambda p, i: (0, 0)),
                  pl.BlockSpec((1, D), lambda p, i: (0, 0)),
                  pl.BlockSpec((1, D), lambda p, i: (0, 0)),
                  pl.BlockSpec((1, D), lambda p, i: (0, 0))],
        out_specs=pl.BlockSpec((BN, D), lambda p, i: (i, 0)),
        out_shape=jax.ShapeDtypeStruct((N, D), jnp.float32),
        scratch_shapes=[pltpu.VMEM((N, D), jnp.float32),
                        pltpu.VMEM((2, D), jnp.float32)])


def kernel(node_feats, edge_index, W, b, Wr, br, gamma, beta):
    N, D = node_feats.shape
    E = edge_index.shape[1]
    EP = E // _NW
    K = 80
    NCH = EP // K
    BN = 2000

    src = edge_index[0]
    dst = edge_index[1]

    degp = _build_degrees(E, N)(src, dst)
    sel = jnp.concatenate(
        [jnp.concatenate([jnp.ones((_NW, 1), jnp.float32),
                          jnp.zeros((_NW, 1), jnp.float32)], axis=1),
         jnp.concatenate([jnp.zeros((_NW, 1), jnp.float32),
                          jnp.ones((_NW, 1), jnp.float32)], axis=1)],
        axis=0)
    h, nc = _build_prep(N, D, BN)(degp.reshape(2 * _NW, N),
                                  node_feats, sel)
    aggp = _build_agg(N, D, E, K)(
        h, src.reshape(_NW, NCH, K), dst.reshape(_NW, NCH, K))
    return _build_main(N, D, BN)(
        aggp[0], aggp[1], nc, node_feats,
        W, b.reshape(1, D), Wr, br.reshape(1, D),
        gamma.reshape(1, D), beta.reshape(1, D))


# speedup vs baseline: 1.0202x; 1.0202x over previous
import jax
import jax.numpy as jnp
from jax import lax
from jax.experimental import pallas as pl
from jax.experimental.pallas import tpu as pltpu
from jax.experimental.pallas import tpu_sc as plsc

_NC = 2
_NS = 16
_NW = _NC * _NS
_L = 16


def _sc_mesh():
    return plsc.VectorSubcoreMesh(
        core_axis_name="c", subcore_axis_name="s",
        num_cores=_NC, num_subcores=_NS)


def _build_degrees(E, N):
    EP = E // _NW

    def body(src_hbm, dst_hbm, out_hbm, idx_a, idx_b, hist0, hist1,
             sema, semb):
        cid = lax.axis_index("c")
        sid = lax.axis_index("s")
        wid = sid * _NC + cid

        pltpu.async_copy(src_hbm.at[pl.ds(wid * EP, EP)], idx_a, sema)
        pltpu.async_copy(dst_hbm.at[pl.ds(wid * EP, EP)], idx_b, semb)

        def zero(i, _):
            hist0[pl.ds(i * _L, _L)] = jnp.zeros((_L,), jnp.float32)
            hist1[pl.ds(i * _L, _L)] = jnp.zeros((_L,), jnp.float32)
            return 0
        lax.fori_loop(0, N // _L, zero, 0)

        ones = jnp.ones((_L,), jnp.float32)
        pltpu.make_async_copy(src_hbm.at[pl.ds(wid * EP, EP)], idx_a,
                              sema).wait()

        def acc0(i, _):
            idx = idx_a[pl.ds(i * _L, _L)]
            plsc.addupdate_scatter(hist0, [idx], ones)
            return 0
        lax.fori_loop(0, EP // _L, acc0, 0)

        pltpu.make_async_copy(dst_hbm.at[pl.ds(wid * EP, EP)], idx_b,
                              semb).wait()

        def acc1(i, _):
            idx = idx_b[pl.ds(i * _L, _L)]
            plsc.addupdate_scatter(hist1, [idx], ones)
            return 0
        lax.fori_loop(0, EP // _L, acc1, 0)

        pltpu.sync_copy(hist0, out_hbm.at[0, wid])
        pltpu.sync_copy(hist1, out_hbm.at[1, wid])

    return pl.kernel(
        body,
        out_type=jax.ShapeDtypeStruct((2, _NW, N), jnp.float32),
        mesh=_sc_mesh(),
        compiler_params=pltpu.CompilerParams(needs_layout_passes=False, use_tc_tiling_on_sc=False),
        scratch_types=[
            pltpu.VMEM((EP,), jnp.int32),
            pltpu.VMEM((EP,), jnp.int32),
            pltpu.VMEM((N,), jnp.float32),
            pltpu.VMEM((N,), jnp.float32),
            pltpu.SemaphoreType.DMA,
            pltpu.SemaphoreType.DMA,
        ])


def _build_agg(N, D, E, K):
    EP = E // _NW
    NCH = EP // K
    RPT = N // _NS

    def body(h_hbm, src_hbm, dst_hbm, out_hbm, sidx, didx, gbuf,
             acc_sh, gsem, ssem):
        cid = lax.axis_index("c")
        sid = lax.axis_index("s")
        wid = sid * _NC + cid

        def z(i, _):
            for bk in range(2):
                for j in range(D // _L):
                    gbuf[bk, i, pl.ds(j * _L, _L)] = jnp.zeros(
                        (_L,), jnp.float32)
            return 0
        lax.fori_loop(0, K, z, 0)
        nfull = RPT // K
        for j in range(nfull):
            pltpu.sync_copy(gbuf.at[0], acc_sh.at[pl.ds(sid * RPT + j * K, K)])
        rem = RPT - nfull * K
        if rem:
            pltpu.sync_copy(gbuf.at[0, pl.ds(0, rem)],
                            acc_sh.at[pl.ds(sid * RPT + nfull * K, rem)])
        plsc.subcore_barrier()

        pltpu.sync_copy(src_hbm.at[wid], sidx)
        pltpu.sync_copy(dst_hbm.at[wid], didx)

        pltpu.async_copy(h_hbm.at[sidx.at[0]], gbuf.at[0], gsem.at[0])
        pltpu.async_copy(h_hbm.at[sidx.at[1]], gbuf.at[1], gsem.at[1])

        def step(c, _):
            p = lax.rem(c, 3)
            q = lax.rem(c + 2, 3)

            @pl.when(c >= 1)
            def _():
                pltpu.make_async_copy(
                    gbuf.at[q], acc_sh.at[didx.at[c - 1]], ssem.at[q]).wait()

            @pl.when(c + 2 < NCH)
            def _():
                pltpu.async_copy(h_hbm.at[sidx.at[c + 2]], gbuf.at[q],
                                 gsem.at[q])

            pltpu.make_async_copy(h_hbm.at[sidx.at[c]], gbuf.at[p],
                                  gsem.at[p]).wait()
            pltpu.async_copy(gbuf.at[p], acc_sh.at[didx.at[c]], ssem.at[p],
                             add=True)
            return 0
        lax.fori_loop(0, NCH, step, 0)
        p_last = (NCH - 1) % 3
        pltpu.make_async_copy(gbuf.at[p_last],
                              acc_sh.at[didx.at[NCH - 1]],
                              ssem.at[p_last]).wait()

        plsc.subcore_barrier()
        pltpu.sync_copy(acc_sh.at[pl.ds(sid * RPT, RPT)],
                        out_hbm.at[cid, pl.ds(sid * RPT, RPT)])

    return pl.kernel(
        body,
        out_type=jax.ShapeDtypeStruct((_NC, N, D), jnp.float32),
        mesh=_sc_mesh(),
        compiler_params=pltpu.CompilerParams(needs_layout_passes=False, use_tc_tiling_on_sc=False),
        scratch_types=[
            pltpu.VMEM((NCH, K), jnp.int32),
            pltpu.VMEM((NCH, K), jnp.int32),
            pltpu.VMEM((3, K, D), jnp.float32),
            pltpu.VMEM_SHARED((N, D), jnp.float32),
            pltpu.SemaphoreType.DMA((3,)),
            pltpu.SemaphoreType.DMA((3,)),
        ])


def _build_prep(N, D, BN):
    G = N // BN

    def body(degp_ref, x_ref, sel_ref, h_ref, nc_ref, nc_s):
        i = pl.program_id(0)

        @pl.when(i == 0)
        def _():
            dcols = lax.dot_general(
                degp_ref[...], sel_ref[...],
                (((0,), (0,)), ((), ())),
                preferred_element_type=jnp.float32)
            nc_s[...] = lax.rsqrt(jnp.where(dcols > 0, dcols, 1.0))

        nc_blk = nc_s[pl.ds(i * BN, BN), :]
        h_ref[...] = x_ref[...] * nc_blk[:, 0:1]
        nc_ref[...] = nc_blk

    return pl.pallas_call(
        body,
        grid=(G,),
        in_specs=[pl.BlockSpec((2 * _NW, N), lambda i: (0, 0)),
                  pl.BlockSpec((BN, D), lambda i: (i, 0)),
                  pl.BlockSpec((2 * _NW, 2), lambda i: (0, 0))],
        out_specs=[pl.BlockSpec((BN, D), lambda i: (i, 0)),
                   pl.BlockSpec((BN, 2), lambda i: (i, 0))],
        out_shape=[jax.ShapeDtypeStruct((N, D), jnp.float32),
                   jax.ShapeDtypeStruct((N, 2), jnp.float32)],
        scratch_shapes=[pltpu.VMEM((N, 2), jnp.float32)])


def _build_main(N, D, BN):
    G = N // BN

    inv_n = 1.0 / N

    def body(a0, a1, nc, x, w_ref, b_ref, wr_ref, br_ref, g_ref, bt_ref,
             out_ref, new_s, stats_s):
        ph = pl.program_id(0)
        j = pl.program_id(1)

        @pl.when(ph == 0)
        def _():
            agg = (a0[...] + a1[...]) * nc[:, 1:2]
            conv = jnp.maximum(
                jnp.dot(agg, w_ref[...], preferred_element_type=jnp.float32)
                + b_ref[...], 0.0)
            res = jnp.maximum(
                jnp.dot(x[...], wr_ref[...],
                        preferred_element_type=jnp.float32)
                + br_ref[...], 0.0)
            new = conv + res
            new_s[pl.ds(j * BN, BN), :] = new
            out_ref[...] = new
            blk = jnp.concatenate(
                [jnp.sum(new, axis=0, keepdims=True),
                 jnp.sum(new * new, axis=0, keepdims=True)], axis=0)

            @pl.when(j == 0)
            def _():
                stats_s[...] = blk

            @pl.when(j > 0)
            def _():
                stats_s[...] = stats_s[...] + blk

        @pl.when(ph == 1)
        def _():
            s = stats_s[...]
            mean = s[0:1, :] * inv_n
            var = s[1:2, :] * inv_n - mean * mean
            out_ref[...] = ((new_s[pl.ds(j * BN, BN), :] - mean)
                            * lax.rsqrt(var + 1e-5)
                            * g_ref[...] + bt_ref[...])

    return pl.pallas_call(
        body,
        grid=(2, G),
        in_specs=[pl.BlockSpec((BN, D), lambda p, i: (i, 0)),
                  pl.BlockSpec((BN, D), lambda p, i: (i, 0)),
                  pl.BlockSpec((BN, 2), lambda p, i: (i, 0)),
                  pl.BlockSpec((BN, D), lambda p, i: (i, 0)),
                  pl.BlockSpec((D, D), lambda p, i: (0, 0)),
                  pl.BlockSpec((1, D), lambda p, i: (0, 0)),
                  pl.BlockSpec((D, D), lambda p, i: (0, 0)),
                  pl.BlockSpec((1, D), lambda p, i: (0, 0)),
                  pl.BlockSpec((1, D), lambda p, i: (0, 0)),
                  pl.BlockSpec((1, D), lambda p, i: (0, 0))],
        out_specs=pl.BlockSpec((BN, D), lambda p, i: (i, 0)),
        out_shape=jax.ShapeDtypeStruct((N, D), jnp.float32),
        scratch_shapes=[pltpu.VMEM((N, D), jnp.float32),
                        pltpu.VMEM((2, D), jnp.float32)])


def kernel(node_feats, edge_index, W, b, Wr, br, gamma, beta):
    N, D = node_feats.shape
    E = edge_index.shape[1]
    EP = E // _NW
    K = 80
    NCH = EP // K
    BN = 2000

    src = edge_index[0]
    dst = edge_index[1]

    degp = _build_degrees(E, N)(src, dst)
    sel = jnp.concatenate(
        [jnp.concatenate([jnp.ones((_NW, 1), jnp.float32),
                          jnp.zeros((_NW, 1), jnp.float32)], axis=1),
         jnp.concatenate([jnp.zeros((_NW, 1), jnp.float32),
                          jnp.ones((_NW, 1), jnp.float32)], axis=1)],
        axis=0)
    h, nc = _build_prep(N, D, BN)(degp.reshape(2 * _NW, N),
                                  node_feats, sel)
    aggp = _build_agg(N, D, E, K)(
        h, src.reshape(_NW, NCH, K), dst.reshape(_NW, NCH, K))
    return _build_main(N, D, BN)(
        aggp[0], aggp[1], nc, node_feats,
        W, b.reshape(1, D), Wr, br.reshape(1, D),
        gamma.reshape(1, D), beta.reshape(1, D))
